# baseline (speedup 1.0000x reference)
import functools

import jax
import jax.numpy as jnp
from jax import lax
from jax.experimental import pallas as pl
from jax.experimental.pallas import tpu as pltpu
from jax.experimental.pallas import tpu_sc as plsc

_VOCAB = 1000
_NC, _NS = 2, 16
_NW = _NC * _NS


def _pmat_body(e_ref, w_ref, p_ref):
    p_ref[...] = lax.dot_general(
        e_ref[...], w_ref[...],
        dimension_numbers=(((1,), (1,)), ((), ())),
        preferred_element_type=jnp.float32)


def _compute_logit_table(e, w):
    return pl.pallas_call(
        _pmat_body,
        out_shape=jax.ShapeDtypeStruct((_VOCAB, _VOCAB), jnp.float32),
    )(e, w)


def _make_sc_gather(total, vocab, chunk):
    b_per_w = total // _NW
    nchunk = b_per_w // chunk
    mesh = plsc.VectorSubcoreMesh(core_axis_name="c", subcore_axis_name="s",
                                  num_cores=_NC, num_subcores=_NS)

    @functools.partial(
        pl.kernel,
        out_type=jax.ShapeDtypeStruct((total, vocab), jnp.float32),
        mesh=mesh,
        compiler_params=pltpu.CompilerParams(use_tc_tiling_on_sc=False),
        scratch_types=[
            pltpu.VMEM((b_per_w,), jnp.int32),
            pltpu.VMEM((chunk, vocab), jnp.float32),
            pltpu.SemaphoreType.DMA,
        ],
    )
    def gather_kernel(p_hbm, idx_hbm, out_hbm, idx_v, rows_v, sem):
        wid = lax.axis_index("s") * _NC + lax.axis_index("c")
        base = wid * b_per_w
        pltpu.sync_copy(idx_hbm.at[pl.ds(base, b_per_w)], idx_v)

        def chunk_body(c, carry):
            pltpu.async_copy(
                p_hbm.at[idx_v.at[pl.ds(c * chunk, chunk)]], rows_v, sem
            ).wait()
            pltpu.sync_copy(rows_v, out_hbm.at[pl.ds(base + c * chunk, chunk)])
            return carry

        lax.fori_loop(0, nchunk, chunk_body, 0)

    return gather_kernel


def kernel(idx, embed_table, head_w):
    b, s = idx.shape
    p = _compute_logit_table(embed_table, head_w)
    flat_idx = idx.reshape(-1).astype(jnp.int32)
    out = _make_sc_gather(b * s, _VOCAB, 32)(p, flat_idx)
    return out.reshape(b, s, _VOCAB)


# speedup vs baseline: 1.0419x; 1.0419x over previous
import functools

import jax
import jax.numpy as jnp
from jax import lax
from jax.experimental import pallas as pl
from jax.experimental.pallas import tpu as pltpu
from jax.experimental.pallas import tpu_sc as plsc

_VOCAB = 1000
_NC, _NS = 2, 16
_NW = _NC * _NS


def _pmat_body(e_ref, w_ref, p_ref):
    p_ref[...] = lax.dot_general(
        e_ref[...], w_ref[...],
        dimension_numbers=(((1,), (1,)), ((), ())),
        preferred_element_type=jnp.float32)


def _compute_logit_table(e, w):
    return pl.pallas_call(
        _pmat_body,
        out_shape=jax.ShapeDtypeStruct((_VOCAB, _VOCAB), jnp.float32),
    )(e, w)


def _make_sc_gather(total, vocab, chunk):
    b_per_w = total // _NW
    nchunk = b_per_w // chunk
    assert nchunk % 2 == 0 and (chunk % 8) == 0
    mesh = plsc.VectorSubcoreMesh(core_axis_name="c", subcore_axis_name="s",
                                  num_cores=_NC, num_subcores=_NS)

    @functools.partial(
        pl.kernel,
        out_type=jax.ShapeDtypeStruct((total, vocab), jnp.float32),
        mesh=mesh,
        compiler_params=pltpu.CompilerParams(use_tc_tiling_on_sc=False),
        scratch_types=[
            pltpu.VMEM((b_per_w,), jnp.int32),
            pltpu.VMEM((chunk, vocab), jnp.float32),
            pltpu.VMEM((chunk, vocab), jnp.float32),
            pltpu.SemaphoreType.DMA,
            pltpu.SemaphoreType.DMA,
            pltpu.SemaphoreType.DMA,
            pltpu.SemaphoreType.DMA,
        ],
    )
    def gather_kernel(p_hbm, idx_hbm, out_hbm, idx_v, rows0, rows1,
                      gs0, gs1, ss0, ss1):
        wid = lax.axis_index("s") * _NC + lax.axis_index("c")
        base = wid * b_per_w
        pltpu.sync_copy(idx_hbm.at[pl.ds(base, b_per_w)], idx_v)

        rows = (rows0, rows1)
        gsem = (gs0, gs1)
        ssem = (ss0, ss1)

        def start_gather(c, j):
            pltpu.async_copy(
                p_hbm.at[idx_v.at[pl.ds(c * chunk, chunk)]], rows[j], gsem[j])

        def wait_dma(j, sem):
            pltpu.make_async_copy(
                rows[j], out_hbm.at[pl.ds(base, chunk)], sem).wait()

        def start_scatter(c, j):
            pltpu.async_copy(
                rows[j], out_hbm.at[pl.ds(base + c * chunk, chunk)], ssem[j])

        start_gather(0, 0)

        def body(t, carry):
            for j in (0, 1):
                c = 2 * t + j
                nj = 1 - j

                @pl.when(c + 1 < nchunk)
                def _():
                    @pl.when(c >= 1)
                    def _():
                        wait_dma(nj, ssem[nj])
                    start_gather(c + 1, nj)

                wait_dma(j, gsem[j])
                start_scatter(c, j)
            return carry

        lax.fori_loop(0, nchunk // 2, body, 0)
        wait_dma(0, ssem[0])
        wait_dma(1, ssem[1])

    return gather_kernel


def kernel(idx, embed_table, head_w):
    b, s = idx.shape
    p = _compute_logit_table(embed_table, head_w)
    flat_idx = idx.reshape(-1).astype(jnp.int32)
    out = _make_sc_gather(b * s, _VOCAB, 40)(p, flat_idx)
    return out.reshape(b, s, _VOCAB)


# speedup vs baseline: 6.9558x; 6.6763x over previous
import functools

import jax
import jax.numpy as jnp
from jax import lax
from jax.experimental import pallas as pl
from jax.experimental.pallas import tpu as pltpu
from jax.experimental.pallas import tpu_sc as plsc

_VOCAB = 1000
_EMB = 8
_NC, _NS = 2, 16
_NW = _NC * _NS
_L = 16


def _make_sc_embed_gather(batch, seq):
    b_per_w = batch // _NW
    mesh = plsc.VectorSubcoreMesh(core_axis_name="c", subcore_axis_name="s",
                                  num_cores=_NC, num_subcores=_NS)

    @functools.partial(
        pl.kernel,
        out_type=jax.ShapeDtypeStruct((seq, _EMB, batch), jnp.float32),
        mesh=mesh,
        compiler_params=pltpu.CompilerParams(use_tc_tiling_on_sc=False,
                                             needs_layout_passes=False),
        scratch_types=[
            pltpu.VMEM((_VOCAB, _EMB), jnp.float32),
            pltpu.VMEM((b_per_w, seq), jnp.int32),
            pltpu.VMEM((seq, _EMB, b_per_w), jnp.float32),
        ],
    )
    def embed_gather(idx_hbm, e_hbm, xt_hbm, e_t, idx_t, xt_t):
        wid = lax.axis_index("s") * _NC + lax.axis_index("c")
        b0 = wid * b_per_w
        pltpu.sync_copy(e_hbm, e_t)
        pltpu.sync_copy(idx_hbm.at[pl.ds(b0, b_per_w)], idx_t)

        lanes = lax.iota(jnp.int32, _L)

        def s_body(s, carry):
            s_vec = jnp.full((_L,), 0, jnp.int32) + s
            for g in range(b_per_w // _L):
                b_vec = lanes + (g * _L)
                row = plsc.load_gather(idx_t, [b_vec, s_vec])
                for d in range(_EMB):
                    d_vec = jnp.full((_L,), d, jnp.int32)
                    vals = plsc.load_gather(e_t, [row, d_vec])
                    xt_t[s, d, pl.ds(g * _L, _L)] = vals
            return carry

        lax.fori_loop(0, seq, s_body, 0)
        pltpu.sync_copy(xt_t, xt_hbm.at[:, :, pl.ds(b0, b_per_w)])

    return embed_gather


def _proj_body(xt_ref, w_ref, t_ref):
    t_ref[0] = lax.dot_general(
        w_ref[...], xt_ref[0],
        dimension_numbers=(((1,), (0,)), ((), ())),
        preferred_element_type=jnp.float32)


def _tc_project(xt, w, seq, batch):
    return pl.pallas_call(
        _proj_body,
        grid=(seq,),
        in_specs=[
            pl.BlockSpec((1, _EMB, batch), lambda s: (s, 0, 0)),
            pl.BlockSpec((_VOCAB, _EMB), lambda s: (0, 0)),
        ],
        out_specs=pl.BlockSpec((1, _VOCAB, batch), lambda s: (s, 0, 0)),
        out_shape=jax.ShapeDtypeStruct((seq, _VOCAB, batch), jnp.float32),
    )(xt, w)


def kernel(idx, embed_table, head_w):
    b, s = idx.shape
    xt = _make_sc_embed_gather(b, s)(idx.astype(jnp.int32), embed_table)
    t = _tc_project(xt, head_w, s, b)
    return t.transpose(2, 0, 1)


# speedup vs baseline: 7.3047x; 1.0502x over previous
import functools

import jax
import jax.numpy as jnp
from jax import lax
from jax.experimental import pallas as pl
from jax.experimental.pallas import tpu as pltpu
from jax.experimental.pallas import tpu_sc as plsc

_VOCAB = 1000
_EMB = 8
_NC, _NS = 2, 16
_NW = _NC * _NS
_L = 16


def _make_sc_embed_gather(batch, seq):
    b_per_w = batch // _NW
    mesh = plsc.VectorSubcoreMesh(core_axis_name="c", subcore_axis_name="s",
                                  num_cores=_NC, num_subcores=_NS)

    @functools.partial(
        pl.kernel,
        out_type=jax.ShapeDtypeStruct((seq, _EMB, batch), jnp.float32),
        mesh=mesh,
        compiler_params=pltpu.CompilerParams(use_tc_tiling_on_sc=False,
                                             needs_layout_passes=False),
        scratch_types=[
            pltpu.VMEM((_VOCAB, _EMB), jnp.float32),
            pltpu.VMEM((b_per_w, seq), jnp.int32),
            pltpu.VMEM((seq, _EMB, b_per_w), jnp.float32),
        ],
    )
    def embed_gather(idx_hbm, e_hbm, xt_hbm, e_t, idx_t, xt_t):
        wid = lax.axis_index("s") * _NC + lax.axis_index("c")
        b0 = wid * b_per_w
        pltpu.sync_copy(e_hbm, e_t)
        pltpu.sync_copy(idx_hbm.at[pl.ds(b0, b_per_w)], idx_t)

        lanes = lax.iota(jnp.int32, _L)

        def s_body(s, carry):
            s_vec = jnp.full((_L,), 0, jnp.int32) + s
            for g in range(b_per_w // _L):
                b_vec = lanes + (g * _L)
                row = plsc.load_gather(idx_t, [b_vec, s_vec])
                for d in range(_EMB):
                    d_vec = jnp.full((_L,), d, jnp.int32)
                    vals = plsc.load_gather(e_t, [row, d_vec])
                    xt_t[s, d, pl.ds(g * _L, _L)] = vals
            return carry

        lax.fori_loop(0, seq, s_body, 0)
        pltpu.sync_copy(xt_t, xt_hbm.at[:, :, pl.ds(b0, b_per_w)])

    return embed_gather


_SB = 2


def _proj_body(xt_ref, w_ref, t_ref):
    for k in range(_SB):
        t_ref[k] = lax.dot_general(
            w_ref[...], xt_ref[k],
            dimension_numbers=(((1,), (0,)), ((), ())),
            preferred_element_type=jnp.float32)


def _tc_project(xt, w, seq, batch):
    return pl.pallas_call(
        _proj_body,
        grid=(seq // _SB,),
        in_specs=[
            pl.BlockSpec((_SB, _EMB, batch), lambda s: (s, 0, 0)),
            pl.BlockSpec((_VOCAB, _EMB), lambda s: (0, 0)),
        ],
        out_specs=pl.BlockSpec((_SB, _VOCAB, batch), lambda s: (s, 0, 0)),
        out_shape=jax.ShapeDtypeStruct((seq, _VOCAB, batch), jnp.float32),
    )(xt, w)


def kernel(idx, embed_table, head_w):
    b, s = idx.shape
    xt = _make_sc_embed_gather(b, s)(idx.astype(jnp.int32), embed_table)
    t = _tc_project(xt, head_w, s, b)
    return t.transpose(2, 0, 1)
